# baseline (speedup 1.0000x reference)
import jax
import jax.numpy as jnp
from jax import lax
from jax.experimental import pallas as pl
from jax.experimental.pallas import tpu as pltpu
from jax.experimental.pallas import tpu_sc as plsc

N_NODES = 10000
D_FEAT = 128
N_EDGES = 320000

NUM_CORES = 2
NUM_SUBCORES = 16
NT = NUM_CORES * NUM_SUBCORES
LANES = 128
ROWS_PER_TILE = 80
E_PAD = NT * ROWS_PER_TILE * LANES
IDX_ROWS = E_PAD // LANES

DST_PAD = 10240
TILE_RANGE = DST_PAD // NT
ACC_LOCAL = TILE_RANGE + 8
TRASH = TILE_RANGE

CAP_ROWS = 96
WCAP = CAP_ROWS * LANES
WL_TOTAL = 4 * NT * WCAP

PB_ROWS = 160
NBLOCKS = IDX_ROWS // PB_ROWS

ACC_ROWS = 10240
ROWS_PER_TILE_ZERO = ACC_ROWS // NUM_SUBCORES


def _build_worklists(edges):
    slabs = []
    for src, dst in edges:
        bucket = dst // TILE_RANGE
        order = jnp.argsort(bucket, stable=True)
        sb = bucket[order]
        packed = (src[order] << 9) | (dst[order] - sb * TILE_RANGE)
        start = jnp.searchsorted(sb, jnp.arange(NT, dtype=sb.dtype))
        pos_in_bucket = jnp.arange(E_PAD, dtype=jnp.int32) - start[sb]
        pos = sb * WCAP + pos_in_bucket
        pos = jnp.where(pos_in_bucket < WCAP, pos, NT * WCAP)
        slab = jnp.full((NT * WCAP,), TRASH, jnp.int32)
        slabs.append(slab.at[pos].set(packed, mode="drop"))
    return jnp.concatenate(slabs).reshape(4 * NT * CAP_ROWS, LANES)


def _sc_layer_body(xc_hbm, xa_hbm, xb_hbm, wl_hbm,
                   out_a, out_b, out_1, out_2,
                   wl, srcbuf, rows, acc, gsem):
    cid = lax.axis_index("c")
    sid = lax.axis_index("s")
    w = cid * NUM_SUBCORES + sid

    z16 = jnp.zeros((16,), jnp.float32)

    convs = ((0, xc_hbm, out_a), (1, xc_hbm, out_b),
             (2, xa_hbm, out_1), (3, xb_hbm, out_2))

    def unpack_src(j, q):
        for g in range(8):
            srcbuf[q, pl.ds(g * 16, 16)] = wl[j, pl.ds(g * 16, 16)] >> 9

    for et, xsrc_hbm, sum_out in convs:
        def fill_z(i, carry):
            r = i >> 3
            col = (i & 7) * 16
            acc[r, pl.ds(col, 16)] = z16
            return carry
        lax.fori_loop(0, ACC_LOCAL * 8, fill_z, 0)

        base = (et * NT + w) * CAP_ROWS
        pltpu.sync_copy(wl_hbm.at[pl.ds(base, CAP_ROWS)], wl)

        unpack_src(0, 0)
        pltpu.async_copy(xsrc_hbm.at[srcbuf.at[0]], rows.at[0], gsem)

        def step(j, carry):
            b = j & 1

            @pl.when(j < CAP_ROWS - 1)
            def _():
                unpack_src(j + 1, 1 - b)

            pltpu.make_async_copy(
                xsrc_hbm.at[pl.ds(0, LANES)], rows.at[b], gsem).wait()

            @pl.when(j < CAP_ROWS - 1)
            def _():
                pltpu.async_copy(
                    xsrc_hbm.at[srcbuf.at[1 - b]], rows.at[1 - b], gsem)

            def grp(g, c):
                dvec = wl[j, pl.ds(g * 16, 16)] & 511
                for l in range(16):
                    d = dvec[l]
                    e = g * 16 + l
                    for f in range(8):
                        plsc.addupdate(
                            acc.at[d, pl.ds(f * 16, 16)],
                            rows[b, e, pl.ds(f * 16, 16)])
                return c
            lax.fori_loop(0, LANES // 16, grp, 0)
            return carry
        lax.fori_loop(0, CAP_ROWS, step, 0)

        pltpu.sync_copy(acc.at[pl.ds(0, TILE_RANGE)],
                        sum_out.at[pl.ds(w * TILE_RANGE, TILE_RANGE)])


def _sc_layer(xc, xa, xb, wl2d):
    fn = pl.kernel(
        _sc_layer_body,
        mesh=plsc.VectorSubcoreMesh(core_axis_name="c", subcore_axis_name="s"),
        out_type=[
            jax.ShapeDtypeStruct((DST_PAD, D_FEAT), jnp.float32),
        ] * 4,
        scratch_types=[
            pltpu.VMEM((CAP_ROWS, LANES), jnp.int32),
            pltpu.VMEM((2, LANES), jnp.int32),
            pltpu.VMEM((2, LANES, D_FEAT), jnp.float32),
            pltpu.VMEM((ACC_LOCAL, D_FEAT), jnp.float32),
            pltpu.SemaphoreType.DMA,
        ],
    )
    return fn(xc, xa, xb, wl2d)


def _sc_deg_body(dI0, dI1, dI2, dI3, deg_out, dst_idx, ones_v, zdbuf, dacc):
    cid = lax.axis_index("c")
    sid = lax.axis_index("s")

    z16 = jnp.zeros((16,), jnp.float32)
    o16 = jnp.ones((16,), jnp.float32)

    def fill_zd(i, carry):
        zdbuf[pl.ds(i * 16, 16)] = z16
        return carry
    lax.fori_loop(0, ROWS_PER_TILE_ZERO // 16, fill_zd, 0)

    def fill_ones(i, carry):
        ones_v[pl.ds(i * 16, 16)] = o16
        return carry
    lax.fori_loop(0, LANES // 16, fill_ones, 0)

    zbase = sid * ROWS_PER_TILE_ZERO
    base_row = (cid * NUM_SUBCORES + sid) * ROWS_PER_TILE

    for et, dI in enumerate((dI0, dI1, dI2, dI3)):
        pltpu.sync_copy(zdbuf, dacc.at[pl.ds(zbase, ROWS_PER_TILE_ZERO)])
        plsc.subcore_barrier()
        pltpu.sync_copy(dI.at[pl.ds(base_row, ROWS_PER_TILE)], dst_idx)

        def step(j, carry):
            pltpu.sync_copy(ones_v, dacc.at[dst_idx.at[j]], add=True)
            return carry
        lax.fori_loop(0, ROWS_PER_TILE, step, 0)
        plsc.subcore_barrier()
        pltpu.sync_copy(
            dacc.at[pl.ds(zbase, ROWS_PER_TILE_ZERO)],
            deg_out.at[pl.ds(et * NUM_CORES * ACC_ROWS + cid * ACC_ROWS
                             + zbase, ROWS_PER_TILE_ZERO)])


def _sc_deg(d0, d1, d2, d3):
    fn = pl.kernel(
        _sc_deg_body,
        mesh=plsc.VectorSubcoreMesh(core_axis_name="c", subcore_axis_name="s"),
        out_type=[
            jax.ShapeDtypeStruct((4 * NUM_CORES * ACC_ROWS,), jnp.float32),
        ],
        scratch_types=[
            pltpu.VMEM((ROWS_PER_TILE, LANES), jnp.int32),
            pltpu.VMEM((LANES,), jnp.float32),
            pltpu.VMEM((ROWS_PER_TILE_ZERO,), jnp.float32),
            pltpu.VMEM_SHARED((ACC_ROWS,), jnp.float32),
        ],
    )
    return fn(d0, d1, d2, d3)[0]


ROW_BLK = 1024
N_BLK = 10


def _tc_layer_body(pa, d0a, d1a, pb, d0b, d1b, p1, d01, d11, p2, d02, d12,
                   xc, xa, xb,
                   wla, wra, ba, wlb, wrb, bb, wl1, wl2, wrc, bc,
                   oc, oa, ob_ref):
    def mean(p, d0, d1):
        deg = jnp.maximum(d0[...] + d1[...], 1.0)
        return p[...] / deg[:, None]

    def lrelu(x):
        return jnp.where(x > 0, x, 0.01 * x)

    m_a = mean(pa, d0a, d1a)
    out_a = (jnp.dot(m_a, wla[...], preferred_element_type=jnp.float32)
             + jnp.dot(xa[...], wra[...], preferred_element_type=jnp.float32)
             + ba[...])
    oa[...] = lrelu(out_a)

    m_b = mean(pb, d0b, d1b)
    out_b = (jnp.dot(m_b, wlb[...], preferred_element_type=jnp.float32)
             + jnp.dot(xb[...], wrb[...], preferred_element_type=jnp.float32)
             + bb[...])
    ob_ref[...] = lrelu(out_b)

    m_1 = mean(p1, d01, d11)
    m_2 = mean(p2, d02, d12)
    out_c = (jnp.dot(m_1, wl1[...], preferred_element_type=jnp.float32)
             + jnp.dot(m_2, wl2[...], preferred_element_type=jnp.float32)
             + jnp.dot(xc[...], wrc[...], preferred_element_type=jnp.float32)
             + bc[...])
    oc[...] = lrelu(out_c)


def _tc_layer(pa, da, pb, db, p1, d1, p2, d2, xc, xa, xb,
              wla, wra, ba, wlb, wrb, bb, wl1, wl2, wrc, bc):
    p_spec = pl.BlockSpec((ROW_BLK, D_FEAT), lambda i: (i, 0))
    d_spec = pl.BlockSpec((ROW_BLK,), lambda i: (i,))
    x_spec = pl.BlockSpec((ROW_BLK, D_FEAT), lambda i: (i, 0))
    w_spec = pl.BlockSpec((D_FEAT, D_FEAT), lambda i: (0, 0))
    b_spec = pl.BlockSpec((1, D_FEAT), lambda i: (0, 0))
    degs = [da, db, d1, d2]
    return pl.pallas_call(
        _tc_layer_body,
        grid=(N_BLK,),
        in_specs=[p_spec, d_spec, d_spec, p_spec, d_spec, d_spec,
                  p_spec, d_spec, d_spec, p_spec, d_spec, d_spec,
                  x_spec, x_spec, x_spec,
                  w_spec, w_spec, b_spec, w_spec, w_spec, b_spec,
                  w_spec, w_spec, w_spec, b_spec],
        out_specs=[x_spec, x_spec, x_spec],
        out_shape=[jax.ShapeDtypeStruct((N_NODES, D_FEAT), jnp.float32)] * 3,
    )(pa, *degs[0], pb, *degs[1], p1, *degs[2], p2, *degs[3],
      xc, xa, xb,
      wla, wra, ba, wlb, wrb, bb, wl1, wl2, wrc, bc)


def _prep_edges(ei):
    pad = E_PAD - N_EDGES
    src = jnp.concatenate(
        [ei[0].astype(jnp.int32), jnp.zeros((pad,), jnp.int32)])
    dst = jnp.concatenate(
        [ei[1].astype(jnp.int32), jnp.full((pad,), N_NODES, jnp.int32)])
    return (dst.reshape(IDX_ROWS, LANES), (src, dst))


def kernel(x_cdr3b, x_tra_peptide, x_trb_peptide, edge_index_c2a,
           edge_index_c2b, edge_index_a2c, edge_index_b2c, params):
    xc, xa, xb = x_cdr3b, x_tra_peptide, x_trb_peptide
    d_c2a, pk_c2a = _prep_edges(edge_index_c2a)
    d_c2b, pk_c2b = _prep_edges(edge_index_c2b)
    d_a2c, pk_a2c = _prep_edges(edge_index_a2c)
    d_b2c, pk_b2c = _prep_edges(edge_index_b2c)

    deg_all = _sc_deg(d_c2a, d_c2b, d_a2c, d_b2c)
    degs = []
    for et in range(4):
        base = et * NUM_CORES * ACC_ROWS
        degs.append((deg_all[base:base + ACC_ROWS],
                     deg_all[base + ACC_ROWS:base + 2 * ACC_ROWS]))

    wl2d = _build_worklists((pk_c2a, pk_c2b, pk_a2c, pk_b2c))

    for lp in params:
        wla, ba, wra = lp["c2a"]
        wlb, bb, wrb = lp["c2b"]
        wl1, b1, wr1 = lp["a2c"]
        wl2, b2, wr2 = lp["b2c"]
        wrc = wr1 + wr2
        bc = (b1 + b2).reshape(1, D_FEAT)

        pa, pb, p1, p2 = _sc_layer(xc, xa, xb, wl2d)

        xc, xa, xb = _tc_layer(
            pa, degs[0], pb, degs[1], p1, degs[2], p2, degs[3], xc, xa, xb,
            wla, wra, ba.reshape(1, D_FEAT),
            wlb, wrb, bb.reshape(1, D_FEAT),
            wl1, wl2, wrc, bc)

    return (xc, xa, xb)


# speedup vs baseline: 6.7031x; 6.7031x over previous
import jax
import jax.numpy as jnp
from jax import lax
from jax.experimental import pallas as pl
from jax.experimental.pallas import tpu as pltpu
from jax.experimental.pallas import tpu_sc as plsc

N_NODES = 10000
D_FEAT = 128
N_EDGES = 320000

NUM_CORES = 2
NUM_SUBCORES = 16
LANES = 128
CHUNK = 64
NBUF = 4
E_PAD = 327680
IDX_ROWS = E_PAD // CHUNK
CHUNKS_PER_TILE = IDX_ROWS // (NUM_CORES * NUM_SUBCORES)
ROWS_PER_TILE = 80

ACC_ROWS = 10240
ZCHUNK = 64
ROWS_PER_TILE_ZERO = ACC_ROWS // NUM_SUBCORES
OUT_ROWS_PER_TILE = 632


PASSES = 4
PASS_ROWS = CHUNKS_PER_TILE // PASSES


def _sc_layer_body(xc_hbm, xa_hbm, xb_hbm,
                   sIa, dIa, sIb, dIb, sI1, dI1, sI2, dI2,
                   out_a, out_b, out_1, out_2,
                   src_idx, dst_idx, rows, gsem, acc):
    cid = lax.axis_index("c")
    sid = lax.axis_index("s")

    z16 = jnp.zeros((16,), jnp.float32)

    zbase = sid * ROWS_PER_TILE_ZERO
    base_row = (cid * NUM_SUBCORES + sid) * CHUNKS_PER_TILE
    ob = jnp.minimum(sid * OUT_ROWS_PER_TILE, N_NODES - OUT_ROWS_PER_TILE)

    convs = ((xc_hbm, sIa, dIa, out_a), (xc_hbm, sIb, dIb, out_b),
             (xa_hbm, sI1, dI1, out_1), (xb_hbm, sI2, dI2, out_2))

    for xsrc_hbm, srcI_hbm, dstI_hbm, sum_out in convs:
        def fill_z(i, carry):
            r = i // 8
            col = (i % 8) * 16
            rows[0, r, pl.ds(col, 16)] = z16
            return carry
        lax.fori_loop(0, CHUNK * 8, fill_z, 0)

        def zero_chunk(k, carry):
            pltpu.sync_copy(rows.at[0],
                            acc.at[pl.ds(zbase + k * ZCHUNK, ZCHUNK)])
            return carry
        lax.fori_loop(0, ROWS_PER_TILE_ZERO // ZCHUNK, zero_chunk, 0)

        plsc.subcore_barrier()

        for p in range(PASSES):
            pbase = base_row + p * PASS_ROWS
            pltpu.sync_copy(srcI_hbm.at[pl.ds(pbase, PASS_ROWS)], src_idx)
            pltpu.sync_copy(dstI_hbm.at[pl.ds(pbase, PASS_ROWS)], dst_idx)
            for jj in range(NBUF - 1):
                pltpu.async_copy(
                    xsrc_hbm.at[src_idx.at[jj]], rows.at[jj], gsem)

            def step(j, carry):
                b = j & (NBUF - 1)
                pltpu.make_async_copy(
                    xsrc_hbm.at[pl.ds(0, CHUNK)], rows.at[b], gsem).wait()

                @pl.when(j + NBUF - 1 < PASS_ROWS)
                def _():
                    pltpu.async_copy(
                        xsrc_hbm.at[src_idx.at[j + NBUF - 1]],
                        rows.at[(j + NBUF - 1) & (NBUF - 1)], gsem)

                pltpu.sync_copy(rows.at[b], acc.at[dst_idx.at[j]], add=True)
                return carry
            lax.fori_loop(0, PASS_ROWS, step, 0)

        plsc.subcore_barrier()

        pltpu.sync_copy(acc.at[pl.ds(ob, OUT_ROWS_PER_TILE)],
                        sum_out.at[cid, pl.ds(ob, OUT_ROWS_PER_TILE)])
        plsc.subcore_barrier()


def _sc_layer(xc, xa, xb, e_c2a, e_c2b, e_a2c, e_b2c):
    fn = pl.kernel(
        _sc_layer_body,
        mesh=plsc.VectorSubcoreMesh(core_axis_name="c", subcore_axis_name="s"),
        out_type=[
            jax.ShapeDtypeStruct((NUM_CORES, N_NODES, D_FEAT), jnp.float32),
        ] * 4,
        scratch_types=[
            pltpu.VMEM((PASS_ROWS, CHUNK), jnp.int32),
            pltpu.VMEM((PASS_ROWS, CHUNK), jnp.int32),
            pltpu.VMEM((NBUF, CHUNK, D_FEAT), jnp.float32),
            pltpu.SemaphoreType.DMA,
            pltpu.VMEM_SHARED((ACC_ROWS, D_FEAT), jnp.float32),
        ],
    )
    return fn(xc, xa, xb, e_c2a[0], e_c2a[1], e_c2b[0], e_c2b[1],
              e_a2c[0], e_a2c[1], e_b2c[0], e_b2c[1])


def _sc_deg_body(dI0, dI1, dI2, dI3, deg_out, dst_idx, ones_v, zdbuf, dacc):
    cid = lax.axis_index("c")
    sid = lax.axis_index("s")

    z16 = jnp.zeros((16,), jnp.float32)
    o16 = jnp.ones((16,), jnp.float32)

    def fill_zd(i, carry):
        zdbuf[pl.ds(i * 16, 16)] = z16
        return carry
    lax.fori_loop(0, ROWS_PER_TILE_ZERO // 16, fill_zd, 0)

    def fill_ones(i, carry):
        ones_v[pl.ds(i * 16, 16)] = o16
        return carry
    lax.fori_loop(0, CHUNK // 16, fill_ones, 0)

    zbase = sid * ROWS_PER_TILE_ZERO
    base_row = (cid * NUM_SUBCORES + sid) * CHUNKS_PER_TILE

    for et, dI in enumerate((dI0, dI1, dI2, dI3)):
        pltpu.sync_copy(zdbuf, dacc.at[pl.ds(zbase, ROWS_PER_TILE_ZERO)])
        plsc.subcore_barrier()
        pltpu.sync_copy(dI.at[pl.ds(base_row, CHUNKS_PER_TILE)], dst_idx)

        def step(j, carry):
            pltpu.sync_copy(ones_v, dacc.at[dst_idx.at[j]], add=True)
            return carry
        lax.fori_loop(0, CHUNKS_PER_TILE, step, 0)
        plsc.subcore_barrier()
        pltpu.sync_copy(
            dacc.at[pl.ds(zbase, ROWS_PER_TILE_ZERO)],
            deg_out.at[pl.ds(et * NUM_CORES * ACC_ROWS + cid * ACC_ROWS
                             + zbase, ROWS_PER_TILE_ZERO)])


def _sc_deg(d0, d1, d2, d3):
    fn = pl.kernel(
        _sc_deg_body,
        mesh=plsc.VectorSubcoreMesh(core_axis_name="c", subcore_axis_name="s"),
        out_type=[
            jax.ShapeDtypeStruct((4 * NUM_CORES * ACC_ROWS,), jnp.float32),
        ],
        scratch_types=[
            pltpu.VMEM((CHUNKS_PER_TILE, CHUNK), jnp.int32),
            pltpu.VMEM((CHUNK,), jnp.float32),
            pltpu.VMEM((ROWS_PER_TILE_ZERO,), jnp.float32),
            pltpu.VMEM_SHARED((ACC_ROWS,), jnp.float32),
        ],
    )
    return fn(d0, d1, d2, d3)[0]


ROW_BLK = 1024
N_BLK = 10


def _tc_layer_body(pa, d0a, d1a, pb, d0b, d1b, p1, d01, d11, p2, d02, d12,
                   xc, xa, xb,
                   wla, wra, ba, wlb, wrb, bb, wl1, wl2, wrc, bc,
                   oc, oa, ob_ref):
    def mean(p, d0, d1):
        s = p[0] + p[1]
        deg = jnp.maximum(d0[...] + d1[...], 1.0)
        return s / deg[:, None]

    def lrelu(x):
        return jnp.where(x > 0, x, 0.01 * x)

    m_a = mean(pa[...], d0a, d1a)
    out_a = (jnp.dot(m_a, wla[...], preferred_element_type=jnp.float32)
             + jnp.dot(xa[...], wra[...], preferred_element_type=jnp.float32)
             + ba[...])
    oa[...] = lrelu(out_a)

    m_b = mean(pb[...], d0b, d1b)
    out_b = (jnp.dot(m_b, wlb[...], preferred_element_type=jnp.float32)
             + jnp.dot(xb[...], wrb[...], preferred_element_type=jnp.float32)
             + bb[...])
    ob_ref[...] = lrelu(out_b)

    m_1 = mean(p1[...], d01, d11)
    m_2 = mean(p2[...], d02, d12)
    out_c = (jnp.dot(m_1, wl1[...], preferred_element_type=jnp.float32)
             + jnp.dot(m_2, wl2[...], preferred_element_type=jnp.float32)
             + jnp.dot(xc[...], wrc[...], preferred_element_type=jnp.float32)
             + bc[...])
    oc[...] = lrelu(out_c)


def _tc_layer(pa, da, pb, db, p1, d1, p2, d2, xc, xa, xb,
              wla, wra, ba, wlb, wrb, bb, wl1, wl2, wrc, bc):
    p_spec = pl.BlockSpec((NUM_CORES, ROW_BLK, D_FEAT), lambda i: (0, i, 0))
    d_spec = pl.BlockSpec((ROW_BLK,), lambda i: (i,))
    x_spec = pl.BlockSpec((ROW_BLK, D_FEAT), lambda i: (i, 0))
    w_spec = pl.BlockSpec((D_FEAT, D_FEAT), lambda i: (0, 0))
    b_spec = pl.BlockSpec((1, D_FEAT), lambda i: (0, 0))
    degs = [da, db, d1, d2]
    return pl.pallas_call(
        _tc_layer_body,
        grid=(N_BLK,),
        in_specs=[p_spec, d_spec, d_spec, p_spec, d_spec, d_spec,
                  p_spec, d_spec, d_spec, p_spec, d_spec, d_spec,
                  x_spec, x_spec, x_spec,
                  w_spec, w_spec, b_spec, w_spec, w_spec, b_spec,
                  w_spec, w_spec, w_spec, b_spec],
        out_specs=[x_spec, x_spec, x_spec],
        out_shape=[jax.ShapeDtypeStruct((N_NODES, D_FEAT), jnp.float32)] * 3,
    )(pa, *degs[0], pb, *degs[1], p1, *degs[2], p2, *degs[3],
      xc, xa, xb,
      wla, wra, ba, wlb, wrb, bb, wl1, wl2, wrc, bc)


def _prep_edges(ei):
    pad = E_PAD - N_EDGES
    src = jnp.concatenate(
        [ei[0].astype(jnp.int32), jnp.zeros((pad,), jnp.int32)])
    dst = jnp.concatenate(
        [ei[1].astype(jnp.int32), jnp.full((pad,), N_NODES, jnp.int32)])
    return src.reshape(IDX_ROWS, CHUNK), dst.reshape(IDX_ROWS, CHUNK)


def kernel(x_cdr3b, x_tra_peptide, x_trb_peptide, edge_index_c2a,
           edge_index_c2b, edge_index_a2c, edge_index_b2c, params):
    xc, xa, xb = x_cdr3b, x_tra_peptide, x_trb_peptide
    e_c2a = _prep_edges(edge_index_c2a)
    e_c2b = _prep_edges(edge_index_c2b)
    e_a2c = _prep_edges(edge_index_a2c)
    e_b2c = _prep_edges(edge_index_b2c)

    deg_all = _sc_deg(e_c2a[1], e_c2b[1], e_a2c[1], e_b2c[1])
    degs = []
    for et in range(4):
        base = et * NUM_CORES * ACC_ROWS
        degs.append((deg_all[base:base + ACC_ROWS],
                     deg_all[base + ACC_ROWS:base + 2 * ACC_ROWS]))

    for lp in params:
        wla, ba, wra = lp["c2a"]
        wlb, bb, wrb = lp["c2b"]
        wl1, b1, wr1 = lp["a2c"]
        wl2, b2, wr2 = lp["b2c"]
        wrc = wr1 + wr2
        bc = (b1 + b2).reshape(1, D_FEAT)

        pa, pb, p1, p2 = _sc_layer(xc, xa, xb, e_c2a, e_c2b, e_a2c, e_b2c)

        xc, xa, xb = _tc_layer(
            pa, degs[0], pb, degs[1], p1, degs[2], p2, degs[3], xc, xa, xb,
            wla, wra, ba.reshape(1, D_FEAT),
            wlb, wrb, bb.reshape(1, D_FEAT),
            wl1, wl2, wrc, bc)

    return (xc, xa, xb)
